# baseline (speedup 1.0000x reference)
import jax
import jax.numpy as jnp
from jax.experimental import pallas as pl


def _loss_kernel(o_ref):
    o_ref[...] = jnp.full((1, 1), jnp.inf, dtype=jnp.float32)


def kernel(y_pred, y_true):
    del y_pred, y_true
    out = pl.pallas_call(
        _loss_kernel,
        out_shape=jax.ShapeDtypeStruct((1, 1), jnp.float32),
    )()
    return out[0, 0]
